# baseline (speedup 1.0000x reference)
import functools

import jax
import jax.numpy as jnp
from jax.experimental import pallas as pl
from jax.experimental.pallas import tpu as pltpu


def _chamfer_kernel(prc, pic, trr, tir, out, t2p_min, psum, pcnt, *, npb, nt):
    j = pl.program_id(1)
    pr = prc[0]
    pi = pic[0]
    tr = trr[0]
    ti = tir[0]
    dr = pr - tr
    di = pi - ti
    dist = dr * dr + di * di
    mag2 = pr * pr + pi * pi
    valid = mag2 > 1e-12
    min_p2t = jnp.min(dist, axis=1, keepdims=True)
    blk_sum = jnp.sum(jnp.where(valid, min_p2t, 0.0))
    blk_cnt = jnp.sum(valid.astype(jnp.float32))
    dist_m = jnp.where(valid, dist, jnp.inf)
    blk_t2p = jnp.min(dist_m, axis=0, keepdims=True)

    @pl.when(j == 0)
    def _init():
        t2p_min[...] = blk_t2p
        psum[0, 0] = blk_sum
        pcnt[0, 0] = blk_cnt

    @pl.when(j != 0)
    def _acc():
        t2p_min[...] = jnp.minimum(t2p_min[...], blk_t2p)
        psum[0, 0] = psum[0, 0] + blk_sum
        pcnt[0, 0] = pcnt[0, 0] + blk_cnt

    @pl.when(j == npb - 1)
    def _final():
        mean_p2t = psum[0, 0] / jnp.maximum(pcnt[0, 0], 1.0)
        mean_t2p = jnp.sum(t2p_min[...]) / nt
        out[...] = jnp.reshape(mean_p2t + mean_t2p, (1, 1, 1))


def _chamfer(pred, target, pb):
    b, np_, _ = pred.shape
    nt = target.shape[1]
    npb = np_ // pb
    prc = pred[..., 0:1]
    pic = pred[..., 1:2]
    trr = jnp.transpose(target[..., 0:1], (0, 2, 1))
    tir = jnp.transpose(target[..., 1:2], (0, 2, 1))
    grid = (b, npb)
    return pl.pallas_call(
        functools.partial(_chamfer_kernel, npb=npb, nt=float(nt)),
        grid=grid,
        in_specs=[
            pl.BlockSpec((1, pb, 1), lambda i, j: (i, j, 0)),
            pl.BlockSpec((1, pb, 1), lambda i, j: (i, j, 0)),
            pl.BlockSpec((1, 1, nt), lambda i, j: (i, 0, 0)),
            pl.BlockSpec((1, 1, nt), lambda i, j: (i, 0, 0)),
        ],
        out_specs=pl.BlockSpec((1, 1, 1), lambda i, j: (i, 0, 0)),
        out_shape=jax.ShapeDtypeStruct((b, 1, 1), jnp.float32),
        scratch_shapes=[
            pltpu.VMEM((1, nt), jnp.float32),
            pltpu.SMEM((1, 1), jnp.float32),
            pltpu.SMEM((1, 1), jnp.float32),
        ],
        compiler_params=pltpu.CompilerParams(
            dimension_semantics=("parallel", "arbitrary"),
        ),
    )(prc, pic, trr, tir)


def kernel(pred_poles, pred_zeros, target_poles_list, target_zeros_list):
    pole_losses = _chamfer(pred_poles, target_poles_list, pb=256)
    zero_losses = _chamfer(pred_zeros, target_zeros_list, pb=256)
    return jnp.mean(pole_losses) + 0.5 * jnp.mean(zero_losses)


# speedup vs baseline: 1.3761x; 1.3761x over previous
import functools

import jax
import jax.numpy as jnp
from jax.experimental import pallas as pl
from jax.experimental.pallas import tpu as pltpu


def _one_chamfer_tile(prc, pic, trr, tir, out, t2p_min, psum, pcnt, j, npb, nt):
    pr = prc[0]
    pi = pic[0]
    tr = trr[0]
    ti = tir[0]
    dr = pr - tr
    di = pi - ti
    dist = dr * dr + di * di
    mag2 = pr * pr + pi * pi
    valid = mag2 > 1e-12
    min_p2t = jnp.min(dist, axis=1, keepdims=True)
    blk_sum = jnp.sum(jnp.where(valid, min_p2t, 0.0))
    blk_cnt = jnp.sum(valid.astype(jnp.float32))
    dist_m = jnp.where(valid, dist, jnp.inf)
    blk_t2p = jnp.min(dist_m, axis=0, keepdims=True)

    @pl.when(j == 0)
    def _init():
        t2p_min[...] = blk_t2p
        psum[0, 0] = blk_sum
        pcnt[0, 0] = blk_cnt

    @pl.when(j != 0)
    def _acc():
        t2p_min[...] = jnp.minimum(t2p_min[...], blk_t2p)
        psum[0, 0] = psum[0, 0] + blk_sum
        pcnt[0, 0] = pcnt[0, 0] + blk_cnt

    @pl.when(j == npb - 1)
    def _final():
        mean_p2t = psum[0, 0] / jnp.maximum(pcnt[0, 0], 1.0)
        mean_t2p = jnp.sum(t2p_min[...]) / nt
        out[...] = jnp.reshape(mean_p2t + mean_t2p, (1, 1, 1))


def _both_kernel(pprc, ppic, ptrr, ptir, zprc, zpic, ztrr, ztir,
                 pole_out, zero_out,
                 p_t2p, p_sum, p_cnt, z_t2p, z_sum, z_cnt,
                 *, npb, ntp, ntz):
    j = pl.program_id(1)
    _one_chamfer_tile(pprc, ppic, ptrr, ptir, pole_out, p_t2p, p_sum, p_cnt,
                      j, npb, float(ntp))
    _one_chamfer_tile(zprc, zpic, ztrr, ztir, zero_out, z_t2p, z_sum, z_cnt,
                      j, npb, float(ntz))


def _split_cols(pred):
    return pred[..., 0:1], pred[..., 1:2]


def _split_rows(target):
    tr = jnp.transpose(target[..., 0:1], (0, 2, 1))
    ti = jnp.transpose(target[..., 1:2], (0, 2, 1))
    return tr, ti


def kernel(pred_poles, pred_zeros, target_poles_list, target_zeros_list):
    b, np_, _ = pred_poles.shape
    ntp = target_poles_list.shape[1]
    ntz = target_zeros_list.shape[1]
    pb = 512
    npb = np_ // pb

    pprc, ppic = _split_cols(pred_poles)
    zprc, zpic = _split_cols(pred_zeros)
    ptrr, ptir = _split_rows(target_poles_list)
    ztrr, ztir = _split_rows(target_zeros_list)

    pred_spec = pl.BlockSpec((1, pb, 1), lambda i, j: (i, j, 0))
    ptgt_spec = pl.BlockSpec((1, 1, ntp), lambda i, j: (i, 0, 0))
    ztgt_spec = pl.BlockSpec((1, 1, ntz), lambda i, j: (i, 0, 0))
    out_spec = pl.BlockSpec((1, 1, 1), lambda i, j: (i, 0, 0))

    pole_losses, zero_losses = pl.pallas_call(
        functools.partial(_both_kernel, npb=npb, ntp=ntp, ntz=ntz),
        grid=(b, npb),
        in_specs=[pred_spec, pred_spec, ptgt_spec, ptgt_spec,
                  pred_spec, pred_spec, ztgt_spec, ztgt_spec],
        out_specs=[out_spec, out_spec],
        out_shape=[jax.ShapeDtypeStruct((b, 1, 1), jnp.float32),
                   jax.ShapeDtypeStruct((b, 1, 1), jnp.float32)],
        scratch_shapes=[
            pltpu.VMEM((1, ntp), jnp.float32),
            pltpu.SMEM((1, 1), jnp.float32),
            pltpu.SMEM((1, 1), jnp.float32),
            pltpu.VMEM((1, ntz), jnp.float32),
            pltpu.SMEM((1, 1), jnp.float32),
            pltpu.SMEM((1, 1), jnp.float32),
        ],
        compiler_params=pltpu.CompilerParams(
            dimension_semantics=("parallel", "arbitrary"),
        ),
    )(pprc, ppic, ptrr, ptir, zprc, zpic, ztrr, ztir)

    return jnp.mean(pole_losses) + 0.5 * jnp.mean(zero_losses)


# speedup vs baseline: 1.4784x; 1.0744x over previous
import functools

import jax
import jax.numpy as jnp
from jax.experimental import pallas as pl
from jax.experimental.pallas import tpu as pltpu


def _one_chamfer_tile(prc, pic, trr, tir, out, t2p_min, psum, pcnt, j, npb, nt):
    pr = prc[0]
    pi = pic[0]
    tr = trr[0]
    ti = tir[0]
    mag2 = pr * pr + pi * pi
    valid = mag2 > 1e-12
    prm = jnp.where(valid, pr, jnp.inf)
    pim = jnp.where(valid, pi, jnp.inf)
    dr = prm - tr
    di = pim - ti
    dist = dr * dr + di * di
    min_p2t = jnp.min(dist, axis=1, keepdims=True)
    blk_sum = jnp.sum(jnp.where(valid, min_p2t, 0.0))
    blk_cnt = jnp.sum(valid.astype(jnp.float32))
    blk_t2p = jnp.min(dist, axis=0, keepdims=True)

    @pl.when(j == 0)
    def _init():
        t2p_min[...] = blk_t2p
        psum[0, 0] = blk_sum
        pcnt[0, 0] = blk_cnt

    @pl.when(j != 0)
    def _acc():
        t2p_min[...] = jnp.minimum(t2p_min[...], blk_t2p)
        psum[0, 0] = psum[0, 0] + blk_sum
        pcnt[0, 0] = pcnt[0, 0] + blk_cnt

    @pl.when(j == npb - 1)
    def _final():
        mean_p2t = psum[0, 0] / jnp.maximum(pcnt[0, 0], 1.0)
        mean_t2p = jnp.sum(t2p_min[...]) / nt
        out[...] = jnp.reshape(mean_p2t + mean_t2p, (1, 1, 1))


def _both_kernel(pprc, ppic, ptrr, ptir, zprc, zpic, ztrr, ztir,
                 pole_out, zero_out,
                 p_t2p, p_sum, p_cnt, z_t2p, z_sum, z_cnt,
                 *, npb, ntp, ntz):
    j = pl.program_id(1)
    _one_chamfer_tile(pprc, ppic, ptrr, ptir, pole_out, p_t2p, p_sum, p_cnt,
                      j, npb, float(ntp))
    _one_chamfer_tile(zprc, zpic, ztrr, ztir, zero_out, z_t2p, z_sum, z_cnt,
                      j, npb, float(ntz))


def _split_cols(pred):
    return pred[..., 0:1], pred[..., 1:2]


def _split_rows(target):
    tr = jnp.transpose(target[..., 0:1], (0, 2, 1))
    ti = jnp.transpose(target[..., 1:2], (0, 2, 1))
    return tr, ti


def kernel(pred_poles, pred_zeros, target_poles_list, target_zeros_list):
    b, np_, _ = pred_poles.shape
    ntp = target_poles_list.shape[1]
    ntz = target_zeros_list.shape[1]
    pb = 1024
    npb = np_ // pb

    pprc, ppic = _split_cols(pred_poles)
    zprc, zpic = _split_cols(pred_zeros)
    ptrr, ptir = _split_rows(target_poles_list)
    ztrr, ztir = _split_rows(target_zeros_list)

    pred_spec = pl.BlockSpec((1, pb, 1), lambda i, j: (i, j, 0))
    ptgt_spec = pl.BlockSpec((1, 1, ntp), lambda i, j: (i, 0, 0))
    ztgt_spec = pl.BlockSpec((1, 1, ntz), lambda i, j: (i, 0, 0))
    out_spec = pl.BlockSpec((1, 1, 1), lambda i, j: (i, 0, 0))

    pole_losses, zero_losses = pl.pallas_call(
        functools.partial(_both_kernel, npb=npb, ntp=ntp, ntz=ntz),
        grid=(b, npb),
        in_specs=[pred_spec, pred_spec, ptgt_spec, ptgt_spec,
                  pred_spec, pred_spec, ztgt_spec, ztgt_spec],
        out_specs=[out_spec, out_spec],
        out_shape=[jax.ShapeDtypeStruct((b, 1, 1), jnp.float32),
                   jax.ShapeDtypeStruct((b, 1, 1), jnp.float32)],
        scratch_shapes=[
            pltpu.VMEM((1, ntp), jnp.float32),
            pltpu.SMEM((1, 1), jnp.float32),
            pltpu.SMEM((1, 1), jnp.float32),
            pltpu.VMEM((1, ntz), jnp.float32),
            pltpu.SMEM((1, 1), jnp.float32),
            pltpu.SMEM((1, 1), jnp.float32),
        ],
        compiler_params=pltpu.CompilerParams(
            dimension_semantics=("parallel", "arbitrary"),
        ),
    )(pprc, ppic, ptrr, ptir, zprc, zpic, ztrr, ztir)

    return jnp.mean(pole_losses) + 0.5 * jnp.mean(zero_losses)
